# baseline (speedup 1.0000x reference)
import functools

import jax
import jax.numpy as jnp
from jax import lax
from jax.experimental import pallas as pl
from jax.experimental.pallas import tpu as pltpu
from jax.experimental.pallas import tpu_sc as plsc

_HIDDEN = 64
_GAMMA = 12.0
_BATCH = 16384
_NC = 2
_NS = 16
_NW = _NC * _NS
_RPW = _BATCH // _NW
_CHUNK = 128
_NCHUNK = _RPW // _CHUNK
_LANES = 16


@functools.partial(
    pl.kernel,
    out_type=jax.ShapeDtypeStruct((_BATCH,), jnp.float32),
    mesh=plsc.VectorSubcoreMesh(core_axis_name="c", subcore_axis_name="s"),
    compiler_params=pltpu.CompilerParams(
        needs_layout_passes=False, use_tc_tiling_on_sc=False),
    scratch_types=[
        pltpu.VMEM((_NCHUNK, _CHUNK), jnp.int32),
        pltpu.VMEM((_NCHUNK, _CHUNK), jnp.int32),
        pltpu.VMEM((_NCHUNK, _CHUNK), jnp.int32),
        pltpu.VMEM((_NCHUNK, _CHUNK, _HIDDEN), jnp.float32),
        pltpu.VMEM((_NCHUNK, _CHUNK, _HIDDEN), jnp.float32),
        pltpu.VMEM((_NCHUNK, _CHUNK, _HIDDEN), jnp.float32),
        pltpu.VMEM((_RPW,), jnp.float32),
        pltpu.SemaphoreType.DMA,
        pltpu.SemaphoreType.DMA,
        pltpu.SemaphoreType.DMA,
        pltpu.SemaphoreType.DMA,
    ],
)
def _transe_score(hidx_hbm, ridx_hbm, tidx_hbm, ent_hbm, rel_hbm, out_hbm,
                  hidx_v, ridx_v, tidx_v, hrows, rrows, trows, out_v,
                  sem0, sem1, sem2, sem3):
    sems = (sem0, sem1, sem2, sem3)
    wid = lax.axis_index("s") * _NC + lax.axis_index("c")
    row0 = wid * _NCHUNK

    pltpu.sync_copy(hidx_hbm.at[pl.ds(row0, _NCHUNK)], hidx_v)
    pltpu.sync_copy(ridx_hbm.at[pl.ds(row0, _NCHUNK)], ridx_v)
    pltpu.sync_copy(tidx_hbm.at[pl.ds(row0, _NCHUNK)], tidx_v)

    copies = []
    for j in range(_NCHUNK):
        copies.append((
            pltpu.async_copy(ent_hbm.at[hidx_v.at[j]], hrows.at[j], sems[j]),
            pltpu.async_copy(rel_hbm.at[ridx_v.at[j]], rrows.at[j], sems[j]),
            pltpu.async_copy(ent_hbm.at[tidx_v.at[j]], trows.at[j], sems[j]),
        ))

    lane = lax.iota(jnp.int32, _LANES)
    for j in range(_NCHUNK):
        for c in copies[j]:
            c.wait()
        h2, r2, t2 = hrows.at[j], rrows.at[j], trows.at[j]

        def group_body(g, carry, h2=h2, r2=r2, t2=t2, j=j):
            samp = g * _LANES + lane

            def d_body(d, acc):
                dv = jnp.full((_LANES,), d, jnp.int32)
                h = plsc.load_gather(h2, [samp, dv])
                r = plsc.load_gather(r2, [samp, dv])
                t = plsc.load_gather(t2, [samp, dv])
                return acc + jnp.abs(h + r - t)

            acc = lax.fori_loop(
                0, _HIDDEN, d_body, jnp.zeros((_LANES,), jnp.float32))
            out_v[pl.ds(j * _CHUNK + g * _LANES, _LANES)] = _GAMMA - acc
            return carry

        lax.fori_loop(0, _CHUNK // _LANES, group_body, 0)

    pltpu.sync_copy(out_v, out_hbm.at[pl.ds(wid * _RPW, _RPW)])


def kernel(sample, entity_embedding, relation_embedding):
    hidx = sample[:, 0].reshape(_NW * _NCHUNK, _CHUNK)
    ridx = sample[:, 1].reshape(_NW * _NCHUNK, _CHUNK)
    tidx = sample[:, 2].reshape(_NW * _NCHUNK, _CHUNK)
    score = _transe_score(hidx, ridx, tidx, entity_embedding,
                          relation_embedding)
    return score.reshape(_BATCH, 1)


# speedup vs baseline: 1.1936x; 1.1936x over previous
import functools

import jax
import jax.numpy as jnp
from jax import lax
from jax.experimental import pallas as pl
from jax.experimental.pallas import tpu as pltpu
from jax.experimental.pallas import tpu_sc as plsc

_HIDDEN = 64
_GAMMA = 12.0
_BATCH = 16384
_NC = 2
_NS = 16
_NW = _NC * _NS
_RPW = _BATCH // _NW
_CHUNK = 128
_NCHUNK = _RPW // _CHUNK
_LANES = 16
_NGRP = _CHUNK // _LANES
_PPAD = 17


@functools.partial(
    pl.kernel,
    out_type=jax.ShapeDtypeStruct((_BATCH,), jnp.float32),
    mesh=plsc.VectorSubcoreMesh(core_axis_name="c", subcore_axis_name="s"),
    compiler_params=pltpu.CompilerParams(
        needs_layout_passes=False, use_tc_tiling_on_sc=False),
    scratch_types=[
        pltpu.VMEM((_RPW, 3), jnp.int32),
        pltpu.VMEM((_NCHUNK, _CHUNK), jnp.int32),
        pltpu.VMEM((_NCHUNK, _CHUNK), jnp.int32),
        pltpu.VMEM((_NCHUNK, _CHUNK), jnp.int32),
        pltpu.VMEM((_NCHUNK, _CHUNK, _HIDDEN), jnp.float32),
        pltpu.VMEM((_NCHUNK, _CHUNK, _HIDDEN), jnp.float32),
        pltpu.VMEM((_NCHUNK, _CHUNK, _HIDDEN), jnp.float32),
        pltpu.VMEM((_LANES, _PPAD), jnp.float32),
        pltpu.VMEM((_RPW,), jnp.float32),
        pltpu.SemaphoreType.DMA,
        pltpu.SemaphoreType.DMA,
        pltpu.SemaphoreType.DMA,
        pltpu.SemaphoreType.DMA,
    ],
)
def _transe_score(sample_hbm, ent_hbm, rel_hbm, out_hbm,
                  samp_v, hidx_v, ridx_v, tidx_v, hrows, rrows, trows,
                  part_v, out_v, sem0, sem1, sem2, sem3):
    sems = (sem0, sem1, sem2, sem3)
    wid = lax.axis_index("s") * _NC + lax.axis_index("c")
    base = wid * _RPW
    lane = lax.iota(jnp.int32, _LANES)

    pltpu.sync_copy(sample_hbm.at[pl.ds(base, _RPW)], samp_v)

    idx_dsts = (hidx_v, ridx_v, tidx_v)
    for j in range(_NCHUNK):
        def split_body(g, carry, j=j):
            row = j * _CHUNK + g * _LANES + lane
            for col in range(3):
                cv = jnp.full((_LANES,), col, jnp.int32)
                idx = plsc.load_gather(samp_v, [row, cv])
                idx_dsts[col][j, pl.ds(g * _LANES, _LANES)] = idx
            return carry
        lax.fori_loop(0, _NGRP, split_body, 0)

    copies = []
    for j in range(_NCHUNK):
        copies.append((
            pltpu.async_copy(ent_hbm.at[hidx_v.at[j]], hrows.at[j], sems[j]),
            pltpu.async_copy(rel_hbm.at[ridx_v.at[j]], rrows.at[j], sems[j]),
            pltpu.async_copy(ent_hbm.at[tidx_v.at[j]], trows.at[j], sems[j]),
        ))

    for j in range(_NCHUNK):
        for c in copies[j]:
            c.wait()
        h2, r2, t2 = hrows.at[j], rrows.at[j], trows.at[j]

        def group_body(g, carry, h2=h2, r2=r2, t2=t2, j=j):
            g0 = g * _LANES
            for s in range(_LANES):
                i = g0 + s
                acc = jnp.zeros((_LANES,), jnp.float32)
                for k in range(_HIDDEN // _LANES):
                    sl = pl.ds(k * _LANES, _LANES)
                    acc = acc + jnp.abs(h2[i, sl] + r2[i, sl] - t2[i, sl])
                sv = jnp.full((_LANES,), s, jnp.int32)
                plsc.store_scatter(part_v, [lane, sv], acc)
            tot = part_v[0, pl.ds(0, _LANES)]
            for q in range(1, _LANES):
                tot = tot + part_v[q, pl.ds(0, _LANES)]
            out_v[pl.ds(j * _CHUNK + g0, _LANES)] = _GAMMA - tot
            return carry

        lax.fori_loop(0, _NGRP, group_body, 0)

    pltpu.sync_copy(out_v, out_hbm.at[pl.ds(base, _RPW)])


def kernel(sample, entity_embedding, relation_embedding):
    score = _transe_score(sample, entity_embedding, relation_embedding)
    return score.reshape(_BATCH, 1)


# speedup vs baseline: 1.5798x; 1.3236x over previous
import functools

import jax
import jax.numpy as jnp
from jax import lax
from jax.experimental import pallas as pl
from jax.experimental.pallas import tpu as pltpu
from jax.experimental.pallas import tpu_sc as plsc

_HIDDEN = 64
_GAMMA = 12.0
_BATCH = 16384
_NC = 2
_NS = 16
_NW = _NC * _NS
_RPW = _BATCH // _NW
_CHUNK = 128
_NCHUNK = _RPW // _CHUNK
_LANES = 16
_NGRP = _CHUNK // _LANES


@functools.partial(
    pl.kernel,
    out_type=jax.ShapeDtypeStruct((_BATCH,), jnp.float32),
    mesh=plsc.VectorSubcoreMesh(core_axis_name="c", subcore_axis_name="s"),
    compiler_params=pltpu.CompilerParams(needs_layout_passes=False),
    scratch_types=[
        pltpu.VMEM((_RPW,), jnp.int32),
        pltpu.VMEM((_RPW,), jnp.int32),
        pltpu.VMEM((_RPW,), jnp.int32),
        pltpu.VMEM((2, _CHUNK, _HIDDEN), jnp.float32),
        pltpu.VMEM((2, _CHUNK, _HIDDEN), jnp.float32),
        pltpu.VMEM((2, _CHUNK, _HIDDEN), jnp.float32),
        pltpu.VMEM((_RPW,), jnp.float32),
        pltpu.SemaphoreType.DMA,
        pltpu.SemaphoreType.DMA,
    ],
)
def _transe_score(hidx_hbm, ridx_hbm, tidx_hbm, ent_hbm, rel_hbm, out_hbm,
                  hidx_v, ridx_v, tidx_v, hrows, rrows, trows, out_v,
                  sem0, sem1):
    sems = (sem0, sem1)
    wid = lax.axis_index("s") * _NC + lax.axis_index("c")
    base = wid * _RPW
    lane = lax.iota(jnp.int32, _LANES)

    pltpu.sync_copy(hidx_hbm.at[pl.ds(base, _RPW)], hidx_v)
    pltpu.sync_copy(ridx_hbm.at[pl.ds(base, _RPW)], ridx_v)
    pltpu.sync_copy(tidx_hbm.at[pl.ds(base, _RPW)], tidx_v)

    def issue_chunk(j):
        b = j % 2
        sem = sems[b]

        def grp(g, carry):
            i0 = j * _CHUNK + g * _LANES
            hvec = hidx_v[pl.ds(i0, _LANES)]
            rvec = ridx_v[pl.ds(i0, _LANES)]
            tvec = tidx_v[pl.ds(i0, _LANES)]
            d0 = g * _LANES
            for s in range(_LANES):
                pltpu.async_copy(
                    ent_hbm.at[pl.ds(hvec[s], 1)],
                    hrows.at[b, pl.ds(d0 + s, 1)], sem)
                pltpu.async_copy(
                    rel_hbm.at[pl.ds(rvec[s], 1)],
                    rrows.at[b, pl.ds(d0 + s, 1)], sem)
                pltpu.async_copy(
                    ent_hbm.at[pl.ds(tvec[s], 1)],
                    trows.at[b, pl.ds(d0 + s, 1)], sem)
            return carry

        lax.fori_loop(0, _NGRP, grp, 0)

    def drain_chunk(j):
        b = j % 2
        sem = sems[b]

        def w(i, carry):
            pltpu.make_async_copy(
                ent_hbm.at[pl.ds(0, 1)], hrows.at[b, pl.ds(0, 1)], sem
            ).wait()
            return carry

        lax.fori_loop(0, 3 * _CHUNK, w, 0)

    def compute_chunk(j):
        b = j % 2
        h2, r2, t2 = hrows.at[b], rrows.at[b], trows.at[b]

        def grp(g, carry):
            g0 = g * _LANES
            outacc = jnp.zeros((_LANES,), jnp.float32)
            for s in range(_LANES):
                i = g0 + s
                acc = jnp.zeros((_LANES,), jnp.float32)
                for k in range(_HIDDEN // _LANES):
                    sl = pl.ds(k * _LANES, _LANES)
                    acc = acc + jnp.abs(h2[i, sl] + r2[i, sl] - t2[i, sl])
                tot = jnp.full((_LANES,), _GAMMA - jnp.sum(acc), jnp.float32)
                outacc = jnp.where(lane == s, tot, outacc)
            out_v[pl.ds(j * _CHUNK + g0, _LANES)] = outacc
            return carry

        lax.fori_loop(0, _NGRP, grp, 0)

    issue_chunk(0)
    for j in range(_NCHUNK):
        if j + 1 < _NCHUNK:
            issue_chunk(j + 1)
        drain_chunk(j)
        compute_chunk(j)

    pltpu.sync_copy(out_v, out_hbm.at[pl.ds(base, _RPW)])


def kernel(sample, entity_embedding, relation_embedding):
    score = _transe_score(sample[:, 0], sample[:, 1], sample[:, 2],
                          entity_embedding, relation_embedding)
    return score.reshape(_BATCH, 1)


# speedup vs baseline: 1.6720x; 1.0583x over previous
import functools

import jax
import jax.numpy as jnp
from jax import lax
from jax.experimental import pallas as pl
from jax.experimental.pallas import tpu as pltpu
from jax.experimental.pallas import tpu_sc as plsc

_HIDDEN = 64
_GAMMA = 12.0
_BATCH = 16384
_NC = 2
_NS = 16
_NW = _NC * _NS
_RPW = _BATCH // _NW
_CHUNK = 128
_NCHUNK = _RPW // _CHUNK
_LANES = 16
_NGRP = _CHUNK // _LANES


@functools.partial(
    pl.kernel,
    out_type=jax.ShapeDtypeStruct((_BATCH,), jnp.float32),
    mesh=plsc.VectorSubcoreMesh(core_axis_name="c", subcore_axis_name="s"),
    compiler_params=pltpu.CompilerParams(needs_layout_passes=False),
    scratch_types=[
        pltpu.VMEM((_RPW,), jnp.int32),
        pltpu.VMEM((_RPW,), jnp.int32),
        pltpu.VMEM((_RPW,), jnp.int32),
        pltpu.VMEM((2, _CHUNK, _HIDDEN), jnp.float32),
        pltpu.VMEM((2, _CHUNK, _HIDDEN), jnp.float32),
        pltpu.VMEM((2, _CHUNK, _HIDDEN), jnp.float32),
        pltpu.VMEM((_RPW,), jnp.float32),
        pltpu.SemaphoreType.DMA,
        pltpu.SemaphoreType.DMA,
    ],
)
def _transe_score(hidx_hbm, ridx_hbm, tidx_hbm, ent_hbm, rel_hbm, out_hbm,
                  hidx_v, ridx_v, tidx_v, hrows, rrows, trows, out_v,
                  sem0, sem1):
    sems = (sem0, sem1)
    wid = lax.axis_index("s") * _NC + lax.axis_index("c")
    base = wid * _RPW
    lane = lax.iota(jnp.int32, _LANES)

    pltpu.sync_copy(hidx_hbm.at[pl.ds(base, _RPW)], hidx_v)
    pltpu.sync_copy(ridx_hbm.at[pl.ds(base, _RPW)], ridx_v)
    pltpu.sync_copy(tidx_hbm.at[pl.ds(base, _RPW)], tidx_v)

    def issue_chunk(j):
        b = j % 2
        sem = sems[b]

        def grp(g, carry):
            i0 = j * _CHUNK + g * _LANES
            hvec = hidx_v[pl.ds(i0, _LANES)]
            rvec = ridx_v[pl.ds(i0, _LANES)]
            tvec = tidx_v[pl.ds(i0, _LANES)]
            d0 = g * _LANES
            for s in range(_LANES):
                pltpu.async_copy(
                    ent_hbm.at[pl.ds(hvec[s], 1)],
                    hrows.at[b, pl.ds(d0 + s, 1)], sem)
                pltpu.async_copy(
                    rel_hbm.at[pl.ds(rvec[s], 1)],
                    rrows.at[b, pl.ds(d0 + s, 1)], sem)
                pltpu.async_copy(
                    ent_hbm.at[pl.ds(tvec[s], 1)],
                    trows.at[b, pl.ds(d0 + s, 1)], sem)
            return carry

        lax.fori_loop(0, _NGRP, grp, 0)

    def drain_chunk(j):
        b = j % 2
        sem = sems[b]
        pltpu.make_async_copy(
            ent_hbm.at[pl.ds(0, _CHUNK)], hrows.at[b], sem).wait()
        pltpu.make_async_copy(
            ent_hbm.at[pl.ds(0, _CHUNK)], rrows.at[b], sem).wait()
        pltpu.make_async_copy(
            ent_hbm.at[pl.ds(0, _CHUNK)], trows.at[b], sem).wait()

    def compute_chunk(j):
        b = j % 2
        h2, r2, t2 = hrows.at[b], rrows.at[b], trows.at[b]

        def grp(g, carry):
            g0 = g * _LANES
            outacc = jnp.zeros((_LANES,), jnp.float32)
            for s in range(_LANES):
                i = g0 + s
                acc = jnp.zeros((_LANES,), jnp.float32)
                for k in range(_HIDDEN // _LANES):
                    sl = pl.ds(k * _LANES, _LANES)
                    acc = acc + jnp.abs(h2[i, sl] + r2[i, sl] - t2[i, sl])
                tot = jnp.full((_LANES,), _GAMMA - jnp.sum(acc), jnp.float32)
                outacc = jnp.where(lane == s, tot, outacc)
            out_v[pl.ds(j * _CHUNK + g0, _LANES)] = outacc
            return carry

        lax.fori_loop(0, _NGRP, grp, 0)

    issue_chunk(0)
    for j in range(_NCHUNK):
        if j + 1 < _NCHUNK:
            issue_chunk(j + 1)
        drain_chunk(j)
        compute_chunk(j)

    pltpu.sync_copy(out_v, out_hbm.at[pl.ds(base, _RPW)])


def kernel(sample, entity_embedding, relation_embedding):
    score = _transe_score(sample[:, 0], sample[:, 1], sample[:, 2],
                          entity_embedding, relation_embedding)
    return score.reshape(_BATCH, 1)
